# Initial kernel scaffold; baseline (speedup 1.0000x reference)
import functools

import jax
import jax.numpy as jnp
from jax import lax
from jax.experimental import pallas as pl
from jax.experimental.pallas import tpu as pltpu
from jax.experimental.pallas import tpu_sc as plsc

N_NODES = 10000
N_EDGES = 320000
BATCH = 64

NC = 2
NS = 16
L = 16
NW = NC * NS
BPW = BATCH // NW
CHUNK = 4000

_mesh = plsc.VectorSubcoreMesh(
    core_axis_name="c", subcore_axis_name="s", num_cores=NC, num_subcores=NS
)


@functools.partial(
    pl.kernel,
    out_type=jax.ShapeDtypeStruct((BATCH, N_NODES), jnp.float32),
    mesh=_mesh,
    scratch_types=[
        pltpu.VMEM((N_NODES,), jnp.float32),
        pltpu.VMEM((N_NODES,), jnp.float32),
        pltpu.VMEM((N_NODES,), jnp.float32),
        pltpu.VMEM((N_NODES,), jnp.float32),
        pltpu.VMEM((CHUNK,), jnp.int32),
        pltpu.VMEM((CHUNK,), jnp.int32),
        pltpu.VMEM((CHUNK,), jnp.int32),
        pltpu.VMEM((CHUNK,), jnp.float32),
        pltpu.VMEM((BATCH, L), jnp.int32),
    ],
)
def _traverse(h_hbm, ekey_hbm, head_hbm, tail_hbm, w_hbm, bkey_hbm, out_hbm,
              h0, h1, o0, o1, ek, eh, et, ew, bk):
    wid = lax.axis_index("s") * NC + lax.axis_index("c")
    b0 = wid * BPW
    b1 = b0 + 1

    pltpu.sync_copy(h_hbm.at[b0], h0)
    pltpu.sync_copy(h_hbm.at[b1], h1)
    pltpu.sync_copy(bkey_hbm, bk)
    bk0 = bk[b0, :]
    bk1 = bk[b1, :]

    zeros = jnp.zeros((L,), jnp.float32)

    def zero_body(j, carry):
        sl = pl.ds(j * L, L)
        o0[sl] = zeros
        o1[sl] = zeros
        return carry

    lax.fori_loop(0, N_NODES // L, zero_body, 0)

    def scatter_max(oref, t16, msg, m):
        old = plsc.load_gather(oref, [t16])
        new = jnp.maximum(old, msg)
        upd = m & (new > old)
        plsc.store_scatter(oref, [t16], new, mask=upd)
        chk = plsc.load_gather(oref, [t16])
        return jnp.any(m & (chk < msg))

    def chunk_body(ci, carry):
        base = ci * CHUNK
        pltpu.sync_copy(ekey_hbm.at[pl.ds(base, CHUNK)], ek)
        pltpu.sync_copy(head_hbm.at[pl.ds(base, CHUNK)], eh)
        pltpu.sync_copy(tail_hbm.at[pl.ds(base, CHUNK)], et)
        pltpu.sync_copy(w_hbm.at[pl.ds(base, CHUNK)], ew)

        def inner(j, icarry):
            sl = pl.ds(j * L, L)
            k16 = ek[sl]
            hd16 = eh[sl]
            t16 = et[sl]
            w16 = ew[sl]
            for bkv, href, oref in ((bk0, h0, o0), (bk1, h1, o1)):
                m = k16 == bkv
                msg = plsc.load_gather(href, [hd16]) * w16
                fail = scatter_max(oref, t16, msg, m)
                lax.while_loop(
                    lambda c: c,
                    lambda c: scatter_max(oref, t16, msg, m),
                    fail,
                )
            return icarry

        return lax.fori_loop(0, CHUNK // L, inner, carry)

    lax.fori_loop(0, N_EDGES // CHUNK, chunk_body, 0)

    pltpu.sync_copy(o0, out_hbm.at[b0])
    pltpu.sync_copy(o1, out_hbm.at[b1])


@jax.jit
def kernel(h_prob, head, tail, rel, ts, edge_weight, r_index, timestamp):
    ekey = rel.astype(jnp.int32) * 4 + ts.astype(jnp.int32)
    bkey = r_index.astype(jnp.int32) * 4 + timestamp.astype(jnp.int32)
    bkey16 = jnp.tile(bkey[:, None], (1, L))
    return _traverse(
        h_prob,
        ekey,
        head.astype(jnp.int32),
        tail.astype(jnp.int32),
        edge_weight,
        bkey16,
    )


# speedup vs baseline: 17.2347x; 17.2347x over previous
import functools

import jax
import jax.numpy as jnp
from jax import lax
from jax.experimental import pallas as pl
from jax.experimental.pallas import tpu as pltpu
from jax.experimental.pallas import tpu_sc as plsc

N_NODES = 10000
N_EDGES = 320000
BATCH = 64

NC = 2
NS = 16
L = 16
NW = NC * NS
BPW = BATCH // NW
CHUNK = 4000

_mesh = plsc.VectorSubcoreMesh(
    core_axis_name="c", subcore_axis_name="s", num_cores=NC, num_subcores=NS
)


@functools.partial(
    pl.kernel,
    out_type=jax.ShapeDtypeStruct((BATCH, N_NODES), jnp.float32),
    mesh=_mesh,
    compiler_params=pltpu.CompilerParams(needs_layout_passes=False),
    scratch_types=[
        pltpu.VMEM((N_NODES,), jnp.float32),
        pltpu.VMEM((N_NODES,), jnp.float32),
        pltpu.VMEM((N_NODES,), jnp.float32),
        pltpu.VMEM((N_NODES,), jnp.float32),
        pltpu.VMEM((CHUNK,), jnp.int32),
        pltpu.VMEM((CHUNK,), jnp.int32),
        pltpu.VMEM((CHUNK,), jnp.int32),
        pltpu.VMEM((CHUNK,), jnp.float32),
        pltpu.VMEM((BATCH, L), jnp.int32),
    ],
)
def _traverse(h_hbm, ekey_hbm, head_hbm, tail_hbm, w_hbm, bkey_hbm, out_hbm,
              h0, h1, o0, o1, ek, eh, et, ew, bk):
    wid = lax.axis_index("s") * NC + lax.axis_index("c")
    b0 = wid * BPW
    b1 = b0 + 1

    pltpu.sync_copy(h_hbm.at[b0], h0)
    pltpu.sync_copy(h_hbm.at[b1], h1)
    pltpu.sync_copy(bkey_hbm, bk)
    bk0 = bk[b0, :]
    bk1 = bk[b1, :]

    zeros = jnp.zeros((L,), jnp.float32)

    def zero_body(j, carry):
        sl = pl.ds(j * L, L)
        o0[sl] = zeros
        o1[sl] = zeros
        return carry

    lax.fori_loop(0, N_NODES // L, zero_body, 0)

    def scatter_max(oref, t16, msg, m):
        old = plsc.load_gather(oref, [t16])
        new = jnp.maximum(old, msg)
        upd = m & (new > old)
        plsc.store_scatter(oref, [t16], new, mask=upd)
        chk = plsc.load_gather(oref, [t16])
        return jnp.any(m & (chk < msg))

    def chunk_body(ci, carry):
        base = ci * CHUNK
        pltpu.sync_copy(ekey_hbm.at[pl.ds(base, CHUNK)], ek)
        pltpu.sync_copy(head_hbm.at[pl.ds(base, CHUNK)], eh)
        pltpu.sync_copy(tail_hbm.at[pl.ds(base, CHUNK)], et)
        pltpu.sync_copy(w_hbm.at[pl.ds(base, CHUNK)], ew)

        def inner(j, icarry):
            sl = pl.ds(j * L, L)
            k16 = ek[sl]
            hd16 = eh[sl]
            t16 = et[sl]
            w16 = ew[sl]
            for bkv, href, oref in ((bk0, h0, o0), (bk1, h1, o1)):
                m = k16 == bkv
                msg = plsc.load_gather(href, [hd16]) * w16
                fail = scatter_max(oref, t16, msg, m)
                lax.while_loop(
                    lambda c: c,
                    lambda c: scatter_max(oref, t16, msg, m),
                    fail,
                )
            return icarry

        return lax.fori_loop(0, CHUNK // L, inner, carry)

    lax.fori_loop(0, N_EDGES // CHUNK, chunk_body, 0)

    pltpu.sync_copy(o0, out_hbm.at[b0])
    pltpu.sync_copy(o1, out_hbm.at[b1])


@jax.jit
def kernel(h_prob, head, tail, rel, ts, edge_weight, r_index, timestamp):
    ekey = rel.astype(jnp.int32) * 4 + ts.astype(jnp.int32)
    bkey = r_index.astype(jnp.int32) * 4 + timestamp.astype(jnp.int32)
    bkey16 = jnp.tile(bkey[:, None], (1, L))
    return _traverse(
        h_prob,
        ekey,
        head.astype(jnp.int32),
        tail.astype(jnp.int32),
        edge_weight,
        bkey16,
    )


# speedup vs baseline: 175.3107x; 10.1720x over previous
import functools

import jax
import jax.numpy as jnp
from jax import lax
from jax.experimental import pallas as pl
from jax.experimental.pallas import tpu as pltpu
from jax.experimental.pallas import tpu_sc as plsc

N_NODES = 10000
N_EDGES = 320000
BATCH = 64

NC = 2
NS = 16
L = 16
NCOMBO = 32
EW = N_EDGES // NS
CH_A = 2000
IDXW = 80
NIDX = CH_A // IDXW
CH_B = 2048
EPS = N_EDGES + NCOMBO * 128 + CH_B

_mesh = plsc.VectorSubcoreMesh(
    core_axis_name="c", subcore_axis_name="s", num_cores=NC, num_subcores=NS
)


@functools.partial(
    pl.kernel,
    out_type=jax.ShapeDtypeStruct((BATCH, N_NODES), jnp.float32),
    mesh=_mesh,
    compiler_params=pltpu.CompilerParams(needs_layout_passes=False),
    scratch_types=[
        pltpu.VMEM((CH_A,), jnp.int32),
        pltpu.VMEM((CH_A,), jnp.int32),
        pltpu.VMEM((CH_A,), jnp.int32),
        pltpu.VMEM((CH_A,), jnp.float32),
        pltpu.VMEM((NIDX, IDXW), jnp.int32),
        pltpu.VMEM((NCOMBO,), jnp.int32),
        pltpu.VMEM((NCOMBO,), jnp.int32),
        pltpu.VMEM((NCOMBO,), jnp.int32),
        pltpu.VMEM((NCOMBO,), jnp.int32),
        pltpu.VMEM((NS, NCOMBO), jnp.int32),
        pltpu.VMEM_SHARED((NS, NCOMBO), jnp.int32),
        pltpu.VMEM_SHARED((EPS,), jnp.int32),
        pltpu.VMEM_SHARED((EPS,), jnp.int32),
        pltpu.VMEM_SHARED((EPS,), jnp.float32),
        pltpu.VMEM((BATCH, L), jnp.int32),
        pltpu.VMEM((N_NODES,), jnp.float32),
        pltpu.VMEM((N_NODES,), jnp.float32),
        pltpu.VMEM((CH_B,), jnp.int32),
        pltpu.VMEM((CH_B,), jnp.int32),
        pltpu.VMEM((CH_B,), jnp.float32),
        pltpu.VMEM((CH_B,), jnp.float32),
        pltpu.VMEM((CH_B,), jnp.int32),
        pltpu.SemaphoreType.DMA,
    ],
)
def _traverse(h_hbm, ekey_hbm, head_hbm, tail_hbm, w_hbm, bkey_hbm, out_hbm,
              kbuf, hd_a, tl_a, w_a, dstbuf, hist, ctr, bases, tot, tbl_v,
              table, sp_head, sp_tail, sp_w, bk_v, hrow, orow,
              hbuf, tbuf, wbuf, msgbuf, tcbuf, dmasem):
    s = lax.axis_index("s")
    k = lax.axis_index("c")
    ebase = s * EW
    iota = lax.broadcasted_iota(jnp.int32, (L,), 0)
    zi = jnp.zeros((L,), jnp.int32)
    zf = jnp.zeros((L,), jnp.float32)

    cal, _ = plsc.scan_count(jnp.full((L,), 7, jnp.int32))
    cbase = jnp.min(cal)

    hist[pl.ds(0, L)] = zi
    hist[pl.ds(L, L)] = zi

    def hist_chunk(ci, carry):
        pltpu.sync_copy(
            ekey_hbm.at[pl.ds(pl.multiple_of(ebase + ci * CH_A, 8), CH_A)], kbuf
        )

        def hist_body(j, c2):
            k16 = kbuf[pl.ds(j * L, L)]
            cnt, last = plsc.scan_count(k16)
            cur = plsc.load_gather(hist, [k16])
            plsc.store_scatter(hist, [k16], cur + (cnt - cbase) + 1, mask=last)
            return c2

        return lax.fori_loop(0, CH_A // L, hist_body, carry)

    lax.fori_loop(0, EW // CH_A, hist_chunk, 0)

    pltpu.sync_copy(hist, table.at[s])
    plsc.subcore_barrier()
    pltpu.sync_copy(table, tbl_v)

    tot_lo = zi
    tot_hi = zi
    my_lo = zi
    my_hi = zi
    for w in range(NS):
        row_lo = tbl_v[w, pl.ds(0, L)]
        row_hi = tbl_v[w, pl.ds(L, L)]
        tot_lo = tot_lo + row_lo
        tot_hi = tot_hi + row_hi
        use = w < s
        my_lo = my_lo + jnp.where(use, row_lo, zi)
        my_hi = my_hi + jnp.where(use, row_hi, zi)

    atot_lo = (tot_lo + 127) & ~127
    atot_hi = (tot_hi + 127) & ~127
    inc_lo = plsc.cumsum(atot_lo)
    inc_hi = plsc.cumsum(atot_hi)
    sum_lo = jnp.max(inc_lo)
    base_lo = inc_lo - atot_lo
    base_hi = inc_hi - atot_hi + sum_lo
    bases[pl.ds(0, L)] = base_lo
    bases[pl.ds(L, L)] = base_hi
    tot[pl.ds(0, L)] = tot_lo
    tot[pl.ds(L, L)] = tot_hi
    ctr[pl.ds(0, L)] = base_lo + my_lo
    ctr[pl.ds(L, L)] = base_hi + my_hi

    def place_chunk(ci, carry):
        abase = pl.multiple_of(ebase + ci * CH_A, 8)
        pltpu.sync_copy(ekey_hbm.at[pl.ds(abase, CH_A)], kbuf)
        pltpu.sync_copy(head_hbm.at[pl.ds(abase, CH_A)], hd_a)
        pltpu.sync_copy(tail_hbm.at[pl.ds(abase, CH_A)], tl_a)
        pltpu.sync_copy(w_hbm.at[pl.ds(abase, CH_A)], w_a)

        def dma_body(d, c2):
            def grp_body(jj, c3):
                j = d * (IDXW // L) + jj
                k16 = kbuf[pl.ds(j * L, L)]
                cnt, last = plsc.scan_count(k16)
                cur = plsc.load_gather(ctr, [k16])
                dst = cur + (cnt - cbase)
                plsc.store_scatter(ctr, [k16], dst + 1, mask=last)
                dstbuf[d, pl.ds(jj * L, L)] = dst
                return c3

            lax.fori_loop(0, IDXW // L, grp_body, 0)
            src = pl.ds(d * IDXW, IDXW)
            idxr = dstbuf.at[d]
            pltpu.async_copy(hd_a.at[src], sp_head.at[idxr], dmasem)
            pltpu.async_copy(tl_a.at[src], sp_tail.at[idxr], dmasem)
            pltpu.async_copy(w_a.at[src], sp_w.at[idxr], dmasem)
            return c2

        lax.fori_loop(0, NIDX, dma_body, 0)

        def drain_body(d, c2):
            src = pl.ds(d * IDXW, IDXW)
            idxr = dstbuf.at[d]
            pltpu.make_async_copy(hd_a.at[src], sp_head.at[idxr], dmasem).wait()
            pltpu.make_async_copy(tl_a.at[src], sp_tail.at[idxr], dmasem).wait()
            pltpu.make_async_copy(w_a.at[src], sp_w.at[idxr], dmasem).wait()
            return c2

        lax.fori_loop(0, NIDX, drain_body, 0)
        return carry

    lax.fori_loop(0, EW // CH_A, place_chunk, 0)
    plsc.subcore_barrier()

    pltpu.sync_copy(bkey_hbm, bk_v)

    for bi in range(BATCH // (NC * NS)):
        b = k * (NS * 2) + s * 2 + bi
        bk16 = bk_v[b, :]
        start = jnp.min(plsc.load_gather(bases, [bk16]))
        cnt = jnp.min(plsc.load_gather(tot, [bk16]))
        pltpu.sync_copy(h_hbm.at[b], hrow)

        def zero_body(j, carry):
            orow[pl.ds(j * L, L)] = zf
            return carry

        lax.fori_loop(0, N_NODES // L, zero_body, 0)

        nch = (cnt + CH_B - 1) // CH_B

        def chunk_body(ci, carry):
            row0 = pl.multiple_of(start + ci * CH_B, 128)
            pltpu.sync_copy(sp_head.at[pl.ds(row0, CH_B)], hbuf)
            pltpu.sync_copy(sp_tail.at[pl.ds(row0, CH_B)], tbuf)
            pltpu.sync_copy(sp_w.at[pl.ds(row0, CH_B)], wbuf)
            limit = cnt - ci * CH_B

            def run_pass(_):
                def main_body(j, c3):
                    sl = pl.ds(j * L, L)
                    m = (j * L + iota) < limit
                    hd = jnp.where(m, hbuf[sl], zi)
                    tl = jnp.where(m, tbuf[sl], zi)
                    msg = plsc.load_gather(hrow, [hd]) * wbuf[sl]
                    msgbuf[sl] = msg
                    tcbuf[sl] = tl
                    old = plsc.load_gather(orow, [tl])
                    new = jnp.maximum(old, msg)
                    plsc.store_scatter(orow, [tl], new, mask=m & (new > old))
                    return c3

                lax.fori_loop(0, CH_B // L, main_body, 0)

                def verify_body(j, facc):
                    sl = pl.ds(j * L, L)
                    m = (j * L + iota) < limit
                    chk = plsc.load_gather(orow, [tcbuf[sl]])
                    return facc | (m & (chk < msgbuf[sl])).astype(jnp.int32)

                fl = lax.fori_loop(0, CH_B // L, verify_body, zi)
                return jnp.max(fl)

            fail = run_pass(0)
            lax.while_loop(lambda x: x > 0, run_pass, fail)
            return carry

        lax.fori_loop(0, nch, chunk_body, 0)
        pltpu.sync_copy(orow, out_hbm.at[b])


@jax.jit
def kernel(h_prob, head, tail, rel, ts, edge_weight, r_index, timestamp):
    ekey = rel.astype(jnp.int32) * 4 + ts.astype(jnp.int32)
    bkey = r_index.astype(jnp.int32) * 4 + timestamp.astype(jnp.int32)
    bkey16 = jnp.tile(bkey[:, None], (1, L))
    return _traverse(
        h_prob,
        ekey,
        head.astype(jnp.int32),
        tail.astype(jnp.int32),
        edge_weight,
        bkey16,
    )
